# baseline (speedup 1.0000x reference)
import jax
import jax.numpy as jnp
from jax.experimental import pallas as pl
from jax.experimental.pallas import tpu as pltpu

_BM = 200


def _gcn_fused_kernel(feat_ref, w_ref, adj_ref, out_ref, fw_ref):
    @pl.when(pl.program_id(0) == 0)
    def _():
        chunk = 1000

        def body(j, _):
            fw_ref[pl.ds(j * chunk, chunk), :] = jnp.dot(
                feat_ref[pl.ds(j * chunk, chunk), :], w_ref[...],
                preferred_element_type=jnp.float32)
            return _

        jax.lax.fori_loop(0, feat_ref.shape[0] // chunk, body, 0)

    out_ref[...] = jax.lax.dot_general(
        adj_ref[...], fw_ref[...],
        dimension_numbers=(((1,), (0,)), ((), ())),
        precision=jax.lax.Precision.DEFAULT,
        preferred_element_type=jnp.float32)


def kernel(features, adj, weight):
    n, d_in = features.shape
    d_out = weight.shape[1]
    feat_bf = features.astype(jnp.bfloat16)
    w_bf = weight.astype(jnp.bfloat16)
    return pl.pallas_call(
        _gcn_fused_kernel,
        grid=(pl.cdiv(n, _BM),),
        in_specs=[
            pl.BlockSpec((n, d_in), lambda i: (0, 0)),
            pl.BlockSpec((d_in, d_out), lambda i: (0, 0)),
            pl.BlockSpec((_BM, n), lambda i: (i, 0)),
        ],
        out_specs=pl.BlockSpec((_BM, d_out), lambda i: (i, 0)),
        out_shape=jax.ShapeDtypeStruct((n, d_out), jnp.float32),
        scratch_shapes=[pltpu.VMEM((n, d_out), jnp.float32)],
        compiler_params=pltpu.CompilerParams(
            dimension_semantics=("arbitrary",)),
    )(feat_bf, w_bf, adj)


# speedup vs baseline: 1.0758x; 1.0758x over previous
import jax
import jax.numpy as jnp
from jax.experimental import pallas as pl
from jax.experimental.pallas import tpu as pltpu

_BM = 400


def _gcn_fused_kernel(feat_ref, w_ref, adj_ref, out_ref, fw_ref):
    @pl.when(pl.program_id(0) == 0)
    def _():
        chunk = 400

        def body(j, carry):
            fw_ref[pl.ds(j * chunk, chunk), :] = jnp.dot(
                feat_ref[pl.ds(j * chunk, chunk), :], w_ref[...],
                preferred_element_type=jnp.float32).astype(jnp.bfloat16)
            return carry

        jax.lax.fori_loop(0, feat_ref.shape[0] // chunk, body, 0)

    out_ref[...] = jax.lax.dot_general(
        adj_ref[...], fw_ref[...],
        dimension_numbers=(((1,), (0,)), ((), ())),
        precision=jax.lax.Precision.DEFAULT,
        preferred_element_type=jnp.float32)


def kernel(features, adj, weight):
    n, d_in = features.shape
    d_out = weight.shape[1]
    feat_bf = features.astype(jnp.bfloat16)
    w_bf = weight.astype(jnp.bfloat16)
    return pl.pallas_call(
        _gcn_fused_kernel,
        grid=(pl.cdiv(n, _BM),),
        in_specs=[
            pl.BlockSpec((n, d_in), lambda i: (0, 0)),
            pl.BlockSpec((d_in, d_out), lambda i: (0, 0)),
            pl.BlockSpec((_BM, n), lambda i: (i, 0)),
        ],
        out_specs=pl.BlockSpec((_BM, d_out), lambda i: (i, 0)),
        out_shape=jax.ShapeDtypeStruct((n, d_out), jnp.float32),
        scratch_shapes=[pltpu.VMEM((n, d_out), jnp.bfloat16)],
        compiler_params=pltpu.CompilerParams(
            dimension_semantics=("arbitrary",)),
    )(feat_bf, w_bf, adj)
